# Initial kernel scaffold; baseline (speedup 1.0000x reference)
import jax
import jax.numpy as jnp
from jax.experimental import pallas as pl


def kernel(x, ln1_g, ln1_b, ln2_g, ln2_b, Wq, bq, Wk, bk, Wv, bv, Wo, bo, Wr, We1, be1, We2, be2):
    raise NotImplementedError("write your pallas kernel here")


# speedup vs baseline: 2.0116x; 2.0116x over previous
import functools
import math

import jax
import jax.numpy as jnp
from jax.experimental import pallas as pl
from jax.experimental.pallas import tpu as pltpu

B, S, D = 1, 2048, 1024
H, DH = 16, 64
E, K, FF = 8, 2, 4096
ALPHA, CZ = 0.01, 0.001
T = B * S

BLK = 256
NB = (T * K) // BLK + E
NS = NB * BLK
NF = 4
FBLK = FF // NF
RB = 256


def _qkv_body(x_ref, g_ref, b_ref, wq_ref, bq_ref, wk_ref, bk_ref,
              wv_ref, bv_ref, q_ref, k_ref, v_ref):
    xx = x_ref[...]
    m = jnp.mean(xx, axis=-1, keepdims=True)
    v = jnp.mean((xx - m) ** 2, axis=-1, keepdims=True)
    hn = (xx - m) * jax.lax.rsqrt(v + 1e-5) * g_ref[...] + b_ref[...]
    q_ref[...] = jnp.dot(hn, wq_ref[...],
                         preferred_element_type=jnp.float32) + bq_ref[...]
    k_ref[...] = jnp.dot(hn, wk_ref[...],
                         preferred_element_type=jnp.float32) + bk_ref[...]
    v_ref[...] = jnp.dot(hn, wv_ref[...],
                         preferred_element_type=jnp.float32) + bv_ref[...]


def _qkv(x2d, ln1_g, ln1_b, Wq, bq, Wk, bk, Wv, bv):
    row = pl.BlockSpec((RB, D), lambda i: (i, 0))
    full = pl.BlockSpec((D, D), lambda i: (0, 0))
    vec = pl.BlockSpec((1, D), lambda i: (0, 0))
    return pl.pallas_call(
        _qkv_body,
        grid=(T // RB,),
        in_specs=[row, vec, vec, full, vec, full, vec, full, vec],
        out_specs=[row, row, row],
        out_shape=[jax.ShapeDtypeStruct((T, D), jnp.float32)] * 3,
        compiler_params=pltpu.CompilerParams(
            dimension_semantics=("arbitrary",),
        ),
    )(x2d, ln1_g.reshape(1, D), ln1_b.reshape(1, D), Wq, bq.reshape(1, D),
      Wk, bk.reshape(1, D), Wv, bv.reshape(1, D))


def _attn_body(q_ref, k_ref, v_ref, o_ref):
    q = q_ref[0]
    k = k_ref[0]
    vv = v_ref[0]
    s = jax.lax.dot_general(q, k, (((1,), (1,)), ((), ())),
                            preferred_element_type=jnp.float32)
    s = s * (1.0 / math.sqrt(DH))
    s = s - jnp.max(s, axis=-1, keepdims=True)
    p = jnp.exp(s)
    p = p / jnp.sum(p, axis=-1, keepdims=True)
    o_ref[0] = jnp.dot(p, vv, preferred_element_type=jnp.float32)


def _attention(qh, kh, vh):
    spec = pl.BlockSpec((1, S, DH), lambda h: (h, 0, 0))
    return pl.pallas_call(
        _attn_body,
        grid=(H,),
        in_specs=[spec, spec, spec],
        out_specs=spec,
        out_shape=jax.ShapeDtypeStruct((H, S, DH), jnp.float32),
        compiler_params=pltpu.CompilerParams(
            dimension_semantics=("arbitrary",),
        ),
    )(qh, kh, vh)


def _post_body(o_ref, x_ref, wo_ref, bo_ref, g2_ref, b2_ref, wr_ref,
               xa_ref, z_ref, i0_ref, i1_ref, w0_ref, w1_ref,
               gate_ref, lse_ref, gate_acc, lse_acc):
    step = pl.program_id(0)
    nstep = pl.num_programs(0)

    @pl.when(step == 0)
    def _init():
        gate_acc[...] = jnp.zeros_like(gate_acc)
        lse_acc[...] = jnp.zeros_like(lse_acc)

    xa = x_ref[...] + jnp.dot(o_ref[...], wo_ref[...],
                              preferred_element_type=jnp.float32) + bo_ref[...]
    xa_ref[...] = xa
    m = jnp.mean(xa, axis=-1, keepdims=True)
    v = jnp.mean((xa - m) ** 2, axis=-1, keepdims=True)
    z = (xa - m) * jax.lax.rsqrt(v + 1e-5) * g2_ref[...] + b2_ref[...]
    z_ref[...] = z

    lg = jnp.dot(z, wr_ref[...], preferred_element_type=jnp.float32)
    lane = jax.lax.broadcasted_iota(jnp.int32, (RB, 128), 1)
    neg = jnp.float32(-1e30)
    lgm = jnp.where(lane < E, lg, neg)
    m1 = jnp.max(lgm, axis=-1, keepdims=True)
    i1 = jnp.min(jnp.where(lgm == m1, lane, 127), axis=-1, keepdims=True)
    lg2 = jnp.where(lane == i1, neg, lgm)
    m2 = jnp.max(lg2, axis=-1, keepdims=True)
    i2 = jnp.min(jnp.where(lg2 == m2, lane, 127), axis=-1, keepdims=True)
    e21 = jnp.exp(m2 - m1)
    w1 = 1.0 / (1.0 + e21)
    w2 = e21 / (1.0 + e21)
    i0_ref[...] = jnp.broadcast_to(i1, (RB, 128))
    i1_ref[...] = jnp.broadcast_to(i2, (RB, 128))
    w0_ref[...] = jnp.broadcast_to(w1, (RB, 128))
    w1_ref[...] = jnp.broadcast_to(w2, (RB, 128))

    ex = jnp.where(lane < E, jnp.exp(lgm - m1), 0.0)
    den = jnp.sum(ex, axis=-1, keepdims=True)
    gate_acc[0:1, :] += jnp.sum(ex / den, axis=0, keepdims=True)
    lse = m1 + jnp.log(den)
    lse_acc[0:1, 0:1] += jnp.sum(lse * lse, axis=0, keepdims=True)[:, 0:1]

    @pl.when(step == nstep - 1)
    def _fin():
        gate_ref[...] = gate_acc[...]
        lse_ref[...] = lse_acc[...]


def _post_attn(o, x2d, Wo, bo, ln2_g, ln2_b, Wr):
    wr_pad = jnp.zeros((D, 128), jnp.float32).at[:, :E].set(Wr.T)
    nstep = T // RB
    outs = pl.pallas_call(
        _post_body,
        grid=(nstep,),
        in_specs=[
            pl.BlockSpec((RB, D), lambda i: (i, 0)),
            pl.BlockSpec((RB, D), lambda i: (i, 0)),
            pl.BlockSpec((D, D), lambda i: (0, 0)),
            pl.BlockSpec((1, D), lambda i: (0, 0)),
            pl.BlockSpec((1, D), lambda i: (0, 0)),
            pl.BlockSpec((1, D), lambda i: (0, 0)),
            pl.BlockSpec((D, 128), lambda i: (0, 0)),
        ],
        out_specs=[
            pl.BlockSpec((RB, D), lambda i: (i, 0)),
            pl.BlockSpec((RB, D), lambda i: (i, 0)),
            pl.BlockSpec((RB, 128), lambda i: (i, 0)),
            pl.BlockSpec((RB, 128), lambda i: (i, 0)),
            pl.BlockSpec((RB, 128), lambda i: (i, 0)),
            pl.BlockSpec((RB, 128), lambda i: (i, 0)),
            pl.BlockSpec((8, 128), lambda i: (0, 0)),
            pl.BlockSpec((8, 128), lambda i: (0, 0)),
        ],
        out_shape=[
            jax.ShapeDtypeStruct((T, D), jnp.float32),
            jax.ShapeDtypeStruct((T, D), jnp.float32),
            jax.ShapeDtypeStruct((T, 128), jnp.int32),
            jax.ShapeDtypeStruct((T, 128), jnp.int32),
            jax.ShapeDtypeStruct((T, 128), jnp.float32),
            jax.ShapeDtypeStruct((T, 128), jnp.float32),
            jax.ShapeDtypeStruct((8, 128), jnp.float32),
            jax.ShapeDtypeStruct((8, 128), jnp.float32),
        ],
        scratch_shapes=[
            pltpu.VMEM((8, 128), jnp.float32),
            pltpu.VMEM((8, 128), jnp.float32),
        ],
        compiler_params=pltpu.CompilerParams(
            dimension_semantics=("arbitrary",),
        ),
    )(o, x2d, Wo, bo.reshape(1, D), ln2_g.reshape(1, D),
      ln2_b.reshape(1, D), wr_pad)
    return outs


def _ffn_body(m_ref, x_ref, w1_ref, b1_ref, w2_ref, b2_ref, o_ref, acc):
    b = pl.program_id(0)
    j = pl.program_id(1)
    used = m_ref[NB]

    @pl.when(b < used)
    def _():
        @pl.when(j == 0)
        def _z():
            acc[...] = jnp.zeros_like(acc)

        h = jnp.dot(x_ref[...], w1_ref[0],
                    preferred_element_type=jnp.float32) + b1_ref[0]
        h = 0.5 * h * (1.0 + jax.lax.erf(h * jnp.float32(0.7071067811865476)))
        acc[...] += jnp.dot(h, w2_ref[0], preferred_element_type=jnp.float32)

        @pl.when(j == NF - 1)
        def _w():
            o_ref[...] = acc[...] + b2_ref[0]


def _expert_ffn(meta, xs, We1, be1, We2, be2):
    def xmap(b, j, m):
        bb = jnp.where(b < m[NB], b, m[NB] - 1)
        return (bb, 0)

    grid_spec = pltpu.PrefetchScalarGridSpec(
        num_scalar_prefetch=1,
        grid=(NB, NF),
        in_specs=[
            pl.BlockSpec((BLK, D), xmap),
            pl.BlockSpec((1, D, FBLK), lambda b, j, m: (m[b], 0, j)),
            pl.BlockSpec((1, 1, FBLK), lambda b, j, m: (m[b] * NF + j, 0, 0)),
            pl.BlockSpec((1, FBLK, D), lambda b, j, m: (m[b], j, 0)),
            pl.BlockSpec((1, 1, D), lambda b, j, m: (m[b], 0, 0)),
        ],
        out_specs=pl.BlockSpec((BLK, D), xmap),
        scratch_shapes=[pltpu.VMEM((BLK, D), jnp.float32)],
    )
    return pl.pallas_call(
        _ffn_body,
        grid_spec=grid_spec,
        out_shape=jax.ShapeDtypeStruct((NS, D), jnp.float32),
        compiler_params=pltpu.CompilerParams(
            dimension_semantics=("arbitrary", "arbitrary"),
        ),
    )(meta, xs, We1, be1.reshape(E * NF, 1, FBLK), We2,
      be2.reshape(E, 1, D))


def kernel(x, ln1_g, ln1_b, ln2_g, ln2_b, Wq, bq, Wk, bk, Wv, bv, Wo, bo,
           Wr, We1, be1, We2, be2):
    x2d = x.reshape(T, D)
    q, k, v = _qkv(x2d, ln1_g, ln1_b, Wq, bq, Wk, bk, Wv, bv)
    qh = q.reshape(S, H, DH).transpose(1, 0, 2)
    kh = k.reshape(S, H, DH).transpose(1, 0, 2)
    vh = v.reshape(S, H, DH).transpose(1, 0, 2)
    oh = _attention(qh, kh, vh)
    o = oh.transpose(1, 0, 2).reshape(T, D)
    (xa, z, i0f, i1f, w0f, w1f, gate_sum, lse2) = _post_attn(
        o, x2d, Wo, bo, ln2_g, ln2_b, Wr)

    i0 = i0f[:, 0]
    i1 = i1f[:, 0]
    w0 = w0f[:, 0]
    w1 = w1f[:, 0]

    e_all = jnp.concatenate([i0, i1])
    onehot = (e_all[:, None] == jnp.arange(E)[None, :]).astype(jnp.int32)
    counts = jnp.sum(onehot, axis=0)
    padc = ((counts + BLK - 1) // BLK) * BLK
    off = jnp.concatenate([jnp.zeros((1,), jnp.int32),
                           jnp.cumsum(padc)[:-1].astype(jnp.int32)])
    rank = jnp.cumsum(onehot, axis=0) - onehot
    my_rank = jnp.take_along_axis(rank, e_all[:, None], axis=1)[:, 0]
    dest_all = off[e_all] + my_rank
    dest0, dest1 = dest_all[:T], dest_all[T:]

    used = jnp.sum(padc) // BLK
    bstart = off // BLK
    barange = jnp.arange(NB, dtype=jnp.int32)
    be_raw = jnp.sum(barange[:, None] >= bstart[None, :], axis=1) - 1
    be_last = be_raw[jnp.maximum(used - 1, 0)]
    be = jnp.where(barange < used, be_raw, be_last).astype(jnp.int32)
    meta = jnp.concatenate([be, used.astype(jnp.int32)[None]])

    xs = jnp.zeros((NS, D), jnp.float32).at[dest0].set(z).at[dest1].set(z)
    os_ = _expert_ffn(meta, xs, We1, be1, We2, be2)
    y2d = xa + w0[:, None] * os_[dest0] + w1[:, None] * os_[dest1]

    gate_mean = gate_sum[0, :E] / T
    l_aux = ALPHA * E * jnp.sum((counts.astype(jnp.float32) / (T * K))
                                * gate_mean)
    cz_lz = CZ * lse2[0, 0] / T
    return y2d.reshape(B, S, D), l_aux, cz_lz
